# baseline (speedup 1.0000x reference)
import functools
import math

import jax
import jax.numpy as jnp
from jax import lax
from jax.experimental import pallas as pl
from jax.experimental.pallas import tpu as pltpu
from jax.experimental.pallas import tpu_sc as plsc

_N = 10000
_E = 320000
_F = 128
_NRBF = 20
_NRBF_PAD = 32
_NB = 3
_CUTOFF = 5.0
_MAXZ = 100
_LN2 = math.log(2.0)
_DELTA = _CUTOFF / (_NRBF - 1)
_COEFF = -0.5 / _DELTA ** 2

_NC = 2
_NS = 16
_NW = _NC * _NS
_EW = _E // _NW
_CHUNK = 40
_NCHUNK = _EW // _CHUNK
_NPAD = 10240
_ROWS_PER_TILE = _NPAD // _NS


def _ssp(x):
    return jnp.maximum(x, 0.0) + jnp.log(1.0 + jnp.exp(-jnp.abs(x))) - _LN2


def _pack_bf16(x):
    xi = lax.bitcast_convert_type(x[:, :_F // 2], jnp.int32)
    yi = lax.bitcast_convert_type(x[:, _F // 2:], jnp.int32)
    lo = lax.shift_right_logical(xi + 32768, 16)
    hi = lax.bitwise_and(yi + 32768, jnp.int32(-65536))
    return lax.bitwise_or(lo, hi)


_NBLK = 1000


def _embed_body(z_ref, emb_ref, wpe_ref, out_ref):
    z = z_ref[...]
    cols = lax.broadcasted_iota(jnp.int32, (_NBLK, _MAXZ), 1)
    oh = (z == cols).astype(jnp.float32)
    x0 = jnp.dot(oh, emb_ref[...], preferred_element_type=jnp.float32)
    out_ref[...] = jnp.dot(x0, wpe_ref[...],
                           preferred_element_type=jnp.float32)


def _embed(Zc, emb, W_pe):
    return pl.pallas_call(
        _embed_body,
        grid=(_N // _NBLK,),
        in_specs=[
            pl.BlockSpec((_NBLK, 1), lambda i: (i, 0)),
            pl.BlockSpec((_MAXZ, _F), lambda i: (0, 0)),
            pl.BlockSpec((_F, _F), lambda i: (0, 0)),
        ],
        out_specs=pl.BlockSpec((_NBLK, _F), lambda i: (i, 0)),
        out_shape=jax.ShapeDtypeStruct((_N, _F), jnp.float32),
    )(Zc, emb, W_pe)


_EBLK = 2560


def _filter_body(rij_ref, wf1_ref, bf1_ref, wf2_ref, bf2_ref,
                 o0_ref, o1_ref, o2_ref):
    r = rij_ref[...]
    rr = r * r
    ones31 = jnp.ones((3, 1), jnp.float32)
    d2 = lax.dot_general(rr, ones31, (((0,), (0,)), ((), ())),
                         preferred_element_type=jnp.float32)
    d = jnp.sqrt(d2)
    ks = lax.broadcasted_iota(jnp.int32, (_EBLK, _NRBF_PAD), 1).astype(
        jnp.float32)
    offs = jnp.where(ks < float(_NRBF), ks * _DELTA, 1.0e6)
    f = jnp.exp(_COEFF * (d - offs) ** 2)
    inside = (d < _CUTOFF).astype(jnp.float32)
    rcut = 0.5 * (jnp.cos(d * (math.pi / _CUTOFF)) + 1.0) * inside
    outs = (o0_ref, o1_ref, o2_ref)
    fb = f.astype(jnp.bfloat16)
    for b in range(_NB):
        s1 = jnp.dot(fb, wf1_ref[b].astype(jnp.bfloat16),
                     preferred_element_type=jnp.float32)
        h = _ssp(s1 + bf1_ref[b]).astype(jnp.bfloat16)
        s2 = jnp.dot(h, wf2_ref[b].astype(jnp.bfloat16),
                     preferred_element_type=jnp.float32)
        outs[b][...] = _pack_bf16((s2 + bf2_ref[b]) * rcut)


def _filters(RijT, Wf1p, bf1r, Wf2, bf2r):
    out_sd = jax.ShapeDtypeStruct((_E, _F // 2), jnp.int32)
    return pl.pallas_call(
        _filter_body,
        grid=(_E // _EBLK,),
        in_specs=[
            pl.BlockSpec((3, _EBLK), lambda i: (0, i)),
            pl.BlockSpec((_NB, _NRBF_PAD, _F), lambda i: (0, 0, 0)),
            pl.BlockSpec((_NB, 1, _F), lambda i: (0, 0, 0)),
            pl.BlockSpec((_NB, _F, _F), lambda i: (0, 0, 0)),
            pl.BlockSpec((_NB, 1, _F), lambda i: (0, 0, 0)),
        ],
        out_specs=[pl.BlockSpec((_EBLK, _F // 2), lambda i: (i, 0))] * _NB,
        out_shape=[out_sd, out_sd, out_sd],
    )(RijT, Wf1p, bf1r, Wf2, bf2r)


def _matmul_body(x_ref, w_ref, out_ref):
    out_ref[...] = jnp.dot(x_ref[...], w_ref[...],
                           preferred_element_type=jnp.float32)


_PBLK = 2000


def _project(x, W):
    return pl.pallas_call(
        _matmul_body,
        grid=(_N // _PBLK,),
        in_specs=[
            pl.BlockSpec((_PBLK, _F), lambda i: (i, 0)),
            pl.BlockSpec((_F, _F), lambda i: (0, 0)),
        ],
        out_specs=pl.BlockSpec((_PBLK, _F), lambda i: (i, 0)),
        out_shape=jax.ShapeDtypeStruct((_N, _F), jnp.float32),
    )(x, W)


def _update_body(agg_ref, x_ref, wo1_ref, wo2_ref, out_ref):
    agg = agg_ref[0] + agg_ref[1]
    h = _ssp(jnp.dot(agg, wo1_ref[...], preferred_element_type=jnp.float32))
    v = jnp.dot(h, wo2_ref[...], preferred_element_type=jnp.float32)
    out_ref[...] = x_ref[...] + v


def _node_update(agg_parts, x, Wo1, Wo2):
    return pl.pallas_call(
        _update_body,
        grid=(_N // _NBLK,),
        in_specs=[
            pl.BlockSpec((_NC, _NBLK, _F), lambda i: (0, i, 0)),
            pl.BlockSpec((_NBLK, _F), lambda i: (i, 0)),
            pl.BlockSpec((_F, _F), lambda i: (0, 0)),
            pl.BlockSpec((_F, _F), lambda i: (0, 0)),
        ],
        out_specs=pl.BlockSpec((_NBLK, _F), lambda i: (i, 0)),
        out_shape=jax.ShapeDtypeStruct((_N, _F), jnp.float32),
    )(agg_parts, x, Wo1, Wo2)


def _sc_agg_body(xf_hbm, w_hbm, idx2_hbm, out_hbm,
                 idx_c, rows_v, w_v, agg_sh,
                 sem_g, sem_w, sem_i, sem_s):
    cid = lax.axis_index("c")
    sid = lax.axis_index("s")
    wid = sid * _NC + cid
    ebase = wid * _EW

    zero16 = jnp.zeros((16,), jnp.float32)

    def _zb(i, carry):
        for k in range(_F // 16):
            rows_v[0, i, pl.ds(k * 16, 16)] = zero16
        return carry

    lax.fori_loop(0, _CHUNK, _zb, 0)
    for j in range(_ROWS_PER_TILE // _CHUNK):
        r0 = sid * _ROWS_PER_TILE + j * _CHUNK
        pltpu.sync_copy(rows_v.at[0], agg_sh.at[pl.ds(r0, _CHUNK)])
    plsc.subcore_barrier()

    def _wait_idx():
        pltpu.make_async_copy(idx2_hbm.at[0], idx_c.at[0], sem_i).wait()

    def _wait_rows(buf):
        pltpu.make_async_copy(xf_hbm.at[pl.ds(0, _CHUNK)],
                              rows_v.at[buf], sem_g).wait()

    def _wait_w(buf):
        pltpu.make_async_copy(w_hbm.at[pl.ds(0, _CHUNK)],
                              w_v.at[buf], sem_w).wait()

    def _wait_scatter(rb):
        pltpu.make_async_copy(rows_v.at[rb],
                              agg_sh.at[idx_c.at[0, 1]], sem_s).wait()

    def _fire_idx(ci, ib):
        pltpu.async_copy(idx2_hbm.at[wid * _NCHUNK + ci], idx_c.at[ib], sem_i)

    def _fire_data(ci, rb, ib):
        base = ebase + ci * _CHUNK
        pltpu.async_copy(xf_hbm.at[idx_c.at[ib, 0]], rows_v.at[rb], sem_g)
        pltpu.async_copy(w_hbm.at[pl.ds(base, _CHUNK)], w_v.at[rb], sem_w)

    himask = jnp.full((16,), -65536, jnp.int32)

    def _unpk(v):
        lo = lax.bitcast_convert_type(lax.shift_left(v, 16), jnp.float32)
        hi = lax.bitcast_convert_type(lax.bitwise_and(v, himask), jnp.float32)
        return lo, hi

    def _mul(rb):
        @plsc.parallel_loop(0, _CHUNK, unroll=2)
        def _body(e):
            for q in range(_F // 32):
                wlo, whi = _unpk(w_v[rb, e, pl.ds(q * 16, 16)])
                slo = pl.ds(q * 16, 16)
                shi = pl.ds(_F // 2 + q * 16, 16)
                rows_v[rb, e, slo] = rows_v[rb, e, slo] * wlo
                rows_v[rb, e, shi] = rows_v[rb, e, shi] * whi

    def _fire_scatter(rb, ib):
        pltpu.async_copy(rows_v.at[rb], agg_sh.at[idx_c.at[ib, 1]],
                         sem_s, add=True)

    _fire_idx(0, 0)
    _wait_idx()
    _fire_data(0, 0, 0)
    _fire_idx(1, 1)
    _wait_idx()
    _fire_data(1, 1, 1)
    _fire_idx(2, 2)
    _wait_idx()
    _fire_data(2, 2, 2)
    _fire_idx(3, 3)

    def _steady(c, carry):
        rb = lax.rem(c, 4)

        @pl.when(c > 0)
        def _():
            _wait_scatter(lax.rem(c + 3, 4))

        @pl.when(c <= _NCHUNK - 4)
        def _():
            _wait_idx()
            _fire_data(c + 3, lax.rem(c + 3, 4), lax.rem(c + 3, 5))

        @pl.when(c <= _NCHUNK - 5)
        def _():
            _fire_idx(c + 4, lax.rem(c + 4, 5))

        _wait_rows(rb)
        _wait_w(rb)
        _mul(rb)
        _fire_scatter(rb, lax.rem(c, 5))
        return carry

    lax.fori_loop(0, _NCHUNK, _steady, 0)
    _wait_scatter((_NCHUNK - 1) % 4)
    plsc.subcore_barrier()

    for j in range(_ROWS_PER_TILE // _CHUNK):
        r0 = sid * _ROWS_PER_TILE + j * _CHUNK
        pltpu.sync_copy(agg_sh.at[pl.ds(r0, _CHUNK)],
                        out_hbm.at[cid, pl.ds(r0, _CHUNK)])


@functools.cache
def _build_sc_agg():
    return functools.partial(
        pl.kernel,
        out_type=jax.ShapeDtypeStruct((_NC, _NPAD, _F), jnp.float32),
        mesh=plsc.VectorSubcoreMesh(core_axis_name="c", subcore_axis_name="s",
                                    num_cores=_NC, num_subcores=_NS),
        scratch_types=[
            pltpu.VMEM((5, 2, _CHUNK), jnp.int32),
            pltpu.VMEM((4, _CHUNK, _F), jnp.float32),
            pltpu.VMEM((4, _CHUNK, _F // 2), jnp.int32),
            pltpu.VMEM_SHARED((_NPAD, _F), jnp.float32),
            pltpu.SemaphoreType.DMA,
            pltpu.SemaphoreType.DMA,
            pltpu.SemaphoreType.DMA,
            pltpu.SemaphoreType.DMA,
        ],
    )(_sc_agg_body)


def _sc_agg(xf, w_edges, idx2):
    return _build_sc_agg()(xf, w_edges, idx2)


def kernel(Z, Rij, idx_i, idx_j, emb, W_pe, W_in2f, Wf1, bf1, Wf2, bf2,
           Wo1, Wo2):
    Zc = Z.astype(jnp.int32).reshape(_N, 1)
    RijT = Rij.T
    Wf1p = jnp.pad(Wf1, ((0, 0), (0, _NRBF_PAD - _NRBF), (0, 0)))
    bf1r = bf1.reshape(_NB, 1, _F)
    bf2r = bf2.reshape(_NB, 1, _F)
    idx2 = jnp.stack([idx_j.astype(jnp.int32).reshape(-1, _CHUNK),
                      idx_i.astype(jnp.int32).reshape(-1, _CHUNK)], axis=1)

    x = _embed(Zc, emb, W_pe)
    W_edges = _filters(RijT, Wf1p, bf1r, Wf2, bf2r)
    for b in range(_NB):
        xf = _project(x, W_in2f[b])
        agg_parts = _sc_agg(xf, W_edges[b], idx2)
        x = _node_update(agg_parts, x, Wo1[b], Wo2[b])
    return x


# speedup vs baseline: 1.5229x; 1.5229x over previous
import functools
import math

import jax
import jax.numpy as jnp
from jax import lax
from jax.experimental import pallas as pl
from jax.experimental.pallas import tpu as pltpu
from jax.experimental.pallas import tpu_sc as plsc

_N = 10000
_E = 320000
_F = 128
_NRBF = 20
_NRBF_PAD = 32
_NB = 3
_CUTOFF = 5.0
_MAXZ = 100
_LN2 = math.log(2.0)
_DELTA = _CUTOFF / (_NRBF - 1)
_COEFF = -0.5 / _DELTA ** 2

_NC = 2
_NS = 16
_NW = _NC * _NS
_EW = _E // _NW
_CHUNK = 40
_NCHUNK = _EW // _CHUNK
_NPAD = 10240
_ROWS_PER_TILE = _NPAD // _NS


def _ssp(x):
    return jnp.maximum(x, 0.0) + jnp.log(1.0 + jnp.exp(-jnp.abs(x))) - _LN2


def _pack_bf16(x):
    xi = lax.bitcast_convert_type(x[:, :_F // 2], jnp.int32)
    yi = lax.bitcast_convert_type(x[:, _F // 2:], jnp.int32)
    lo = lax.shift_right_logical(xi + 32768, 16)
    hi = lax.bitwise_and(yi + 32768, jnp.int32(-65536))
    return lax.bitwise_or(lo, hi)


_NBLK = 1000


def _embed_body(z_ref, emb_ref, wpe_ref, out_ref):
    z = z_ref[...]
    cols = lax.broadcasted_iota(jnp.int32, (_NBLK, _MAXZ), 1)
    oh = (z == cols).astype(jnp.float32)
    x0 = jnp.dot(oh, emb_ref[...], preferred_element_type=jnp.float32)
    out_ref[...] = jnp.dot(x0, wpe_ref[...],
                           preferred_element_type=jnp.float32)


def _embed(Zc, emb, W_pe):
    return pl.pallas_call(
        _embed_body,
        grid=(_N // _NBLK,),
        in_specs=[
            pl.BlockSpec((_NBLK, 1), lambda i: (i, 0)),
            pl.BlockSpec((_MAXZ, _F), lambda i: (0, 0)),
            pl.BlockSpec((_F, _F), lambda i: (0, 0)),
        ],
        out_specs=pl.BlockSpec((_NBLK, _F), lambda i: (i, 0)),
        out_shape=jax.ShapeDtypeStruct((_N, _F), jnp.float32),
    )(Zc, emb, W_pe)


_EBLK = 2560


def _filter_body(rij_ref, wf1_ref, bf1_ref, wf2_ref, bf2_ref,
                 o0_ref, o1_ref, o2_ref):
    r = rij_ref[...]
    rr = r * r
    d2 = jnp.sum(rr, axis=0, keepdims=True)
    d = jnp.sqrt(d2)
    ks = lax.broadcasted_iota(jnp.int32, (_NRBF_PAD, _EBLK), 0).astype(
        jnp.float32)
    offs = jnp.where(ks < float(_NRBF), ks * _DELTA, 1.0e6)
    f = jnp.exp(_COEFF * (d - offs) ** 2)
    inside = (d < _CUTOFF).astype(jnp.float32)
    rcut = 0.5 * (jnp.cos(d * (math.pi / _CUTOFF)) + 1.0) * inside
    outs = (o0_ref, o1_ref, o2_ref)
    fb = f.astype(jnp.bfloat16)
    half = _F // 2
    for b in range(_NB):
        s1 = jnp.dot(wf1_ref[b].astype(jnp.bfloat16), fb,
                     preferred_element_type=jnp.float32)
        h = _ssp(s1 + bf1_ref[b]).astype(jnp.bfloat16)
        s2 = jnp.dot(wf2_ref[b].astype(jnp.bfloat16), h,
                     preferred_element_type=jnp.float32)
        wij = (s2 + bf2_ref[b]) * rcut
        xi = lax.bitcast_convert_type(wij[:half, :], jnp.int32)
        yi = lax.bitcast_convert_type(wij[half:, :], jnp.int32)
        lo = lax.shift_right_logical(xi + 32768, 16)
        hi = lax.bitwise_and(yi + 32768, jnp.int32(-65536))
        outs[b][...] = jnp.transpose(lax.bitwise_or(lo, hi))


def _filters(RijT, Wf1T, bf1c, Wf2T, bf2c):
    out_sd = jax.ShapeDtypeStruct((_E, _F // 2), jnp.int32)
    return pl.pallas_call(
        _filter_body,
        grid=(_E // _EBLK,),
        in_specs=[
            pl.BlockSpec((3, _EBLK), lambda i: (0, i)),
            pl.BlockSpec((_NB, _F, _NRBF_PAD), lambda i: (0, 0, 0)),
            pl.BlockSpec((_NB, _F, 1), lambda i: (0, 0, 0)),
            pl.BlockSpec((_NB, _F, _F), lambda i: (0, 0, 0)),
            pl.BlockSpec((_NB, _F, 1), lambda i: (0, 0, 0)),
        ],
        out_specs=[pl.BlockSpec((_EBLK, _F // 2), lambda i: (i, 0))] * _NB,
        out_shape=[out_sd, out_sd, out_sd],
    )(RijT, Wf1T, bf1c, Wf2T, bf2c)


def _matmul_body(x_ref, w_ref, out_ref):
    out_ref[...] = jnp.dot(x_ref[...], w_ref[...],
                           preferred_element_type=jnp.float32)


_PBLK = 2000


def _project(x, W):
    return pl.pallas_call(
        _matmul_body,
        grid=(_N // _PBLK,),
        in_specs=[
            pl.BlockSpec((_PBLK, _F), lambda i: (i, 0)),
            pl.BlockSpec((_F, _F), lambda i: (0, 0)),
        ],
        out_specs=pl.BlockSpec((_PBLK, _F), lambda i: (i, 0)),
        out_shape=jax.ShapeDtypeStruct((_N, _F), jnp.float32),
    )(x, W)


def _update_body(agg_ref, x_ref, wo1_ref, wo2_ref, out_ref):
    agg = agg_ref[0] + agg_ref[1]
    h = _ssp(jnp.dot(agg, wo1_ref[...], preferred_element_type=jnp.float32))
    v = jnp.dot(h, wo2_ref[...], preferred_element_type=jnp.float32)
    out_ref[...] = x_ref[...] + v


def _node_update(agg_parts, x, Wo1, Wo2):
    return pl.pallas_call(
        _update_body,
        grid=(_N // _NBLK,),
        in_specs=[
            pl.BlockSpec((_NC, _NBLK, _F), lambda i: (0, i, 0)),
            pl.BlockSpec((_NBLK, _F), lambda i: (i, 0)),
            pl.BlockSpec((_F, _F), lambda i: (0, 0)),
            pl.BlockSpec((_F, _F), lambda i: (0, 0)),
        ],
        out_specs=pl.BlockSpec((_NBLK, _F), lambda i: (i, 0)),
        out_shape=jax.ShapeDtypeStruct((_N, _F), jnp.float32),
    )(agg_parts, x, Wo1, Wo2)


def _sc_agg_body(xf_hbm, w_hbm, idx2_hbm, out_hbm,
                 idx_c, rows_v, w_v, agg_sh,
                 sem_g, sem_w, sem_i, sem_s):
    cid = lax.axis_index("c")
    sid = lax.axis_index("s")
    wid = sid * _NC + cid
    ebase = wid * _EW

    zero16 = jnp.zeros((16,), jnp.float32)

    def _zb(i, carry):
        for k in range(_F // 16):
            rows_v[0, i, pl.ds(k * 16, 16)] = zero16
        return carry

    lax.fori_loop(0, _CHUNK, _zb, 0)
    for j in range(_ROWS_PER_TILE // _CHUNK):
        r0 = sid * _ROWS_PER_TILE + j * _CHUNK
        pltpu.sync_copy(rows_v.at[0], agg_sh.at[pl.ds(r0, _CHUNK)])
    plsc.subcore_barrier()

    def _wait_idx():
        pltpu.make_async_copy(idx2_hbm.at[0], idx_c.at[0], sem_i).wait()

    def _wait_rows(buf):
        pltpu.make_async_copy(xf_hbm.at[pl.ds(0, _CHUNK)],
                              rows_v.at[buf], sem_g).wait()

    def _wait_w(buf):
        pltpu.make_async_copy(w_hbm.at[pl.ds(0, _CHUNK)],
                              w_v.at[buf], sem_w).wait()

    def _wait_scatter(rb):
        pltpu.make_async_copy(rows_v.at[rb],
                              agg_sh.at[idx_c.at[0, 1]], sem_s).wait()

    def _fire_idx(ci, ib):
        pltpu.async_copy(idx2_hbm.at[wid * _NCHUNK + ci], idx_c.at[ib], sem_i)

    def _fire_data(ci, rb, ib):
        base = ebase + ci * _CHUNK
        pltpu.async_copy(xf_hbm.at[idx_c.at[ib, 0]], rows_v.at[rb], sem_g)
        pltpu.async_copy(w_hbm.at[pl.ds(base, _CHUNK)], w_v.at[rb], sem_w)

    himask = jnp.full((16,), -65536, jnp.int32)

    def _unpk(v):
        lo = lax.bitcast_convert_type(lax.shift_left(v, 16), jnp.float32)
        hi = lax.bitcast_convert_type(lax.bitwise_and(v, himask), jnp.float32)
        return lo, hi

    def _mul(rb):
        @plsc.parallel_loop(0, _CHUNK, unroll=2)
        def _body(e):
            for q in range(_F // 32):
                wlo, whi = _unpk(w_v[rb, e, pl.ds(q * 16, 16)])
                slo = pl.ds(q * 16, 16)
                shi = pl.ds(_F // 2 + q * 16, 16)
                rows_v[rb, e, slo] = rows_v[rb, e, slo] * wlo
                rows_v[rb, e, shi] = rows_v[rb, e, shi] * whi

    def _fire_scatter(rb, ib):
        pltpu.async_copy(rows_v.at[rb], agg_sh.at[idx_c.at[ib, 1]],
                         sem_s, add=True)

    _fire_idx(0, 0)
    _wait_idx()
    _fire_data(0, 0, 0)
    _fire_idx(1, 1)
    _wait_idx()
    _fire_data(1, 1, 1)
    _fire_idx(2, 2)
    _wait_idx()
    _fire_data(2, 2, 2)
    _fire_idx(3, 3)

    def _steady(c, carry):
        rb = lax.rem(c, 4)

        @pl.when(c > 0)
        def _():
            _wait_scatter(lax.rem(c + 3, 4))

        @pl.when(c <= _NCHUNK - 4)
        def _():
            _wait_idx()
            _fire_data(c + 3, lax.rem(c + 3, 4), lax.rem(c + 3, 5))

        @pl.when(c <= _NCHUNK - 5)
        def _():
            _fire_idx(c + 4, lax.rem(c + 4, 5))

        _wait_rows(rb)
        _wait_w(rb)
        _mul(rb)
        _fire_scatter(rb, lax.rem(c, 5))
        return carry

    lax.fori_loop(0, _NCHUNK, _steady, 0)
    _wait_scatter((_NCHUNK - 1) % 4)
    plsc.subcore_barrier()

    for j in range(_ROWS_PER_TILE // _CHUNK):
        r0 = sid * _ROWS_PER_TILE + j * _CHUNK
        pltpu.sync_copy(agg_sh.at[pl.ds(r0, _CHUNK)],
                        out_hbm.at[cid, pl.ds(r0, _CHUNK)])


@functools.cache
def _build_sc_agg():
    return functools.partial(
        pl.kernel,
        out_type=jax.ShapeDtypeStruct((_NC, _NPAD, _F), jnp.float32),
        mesh=plsc.VectorSubcoreMesh(core_axis_name="c", subcore_axis_name="s",
                                    num_cores=_NC, num_subcores=_NS),
        scratch_types=[
            pltpu.VMEM((5, 2, _CHUNK), jnp.int32),
            pltpu.VMEM((4, _CHUNK, _F), jnp.float32),
            pltpu.VMEM((4, _CHUNK, _F // 2), jnp.int32),
            pltpu.VMEM_SHARED((_NPAD, _F), jnp.float32),
            pltpu.SemaphoreType.DMA,
            pltpu.SemaphoreType.DMA,
            pltpu.SemaphoreType.DMA,
            pltpu.SemaphoreType.DMA,
        ],
    )(_sc_agg_body)


def _sc_agg(xf, w_edges, idx2):
    return _build_sc_agg()(xf, w_edges, idx2)


def kernel(Z, Rij, idx_i, idx_j, emb, W_pe, W_in2f, Wf1, bf1, Wf2, bf2,
           Wo1, Wo2):
    Zc = Z.astype(jnp.int32).reshape(_N, 1)
    RijT = Rij.T
    Wf1T = jnp.transpose(
        jnp.pad(Wf1, ((0, 0), (0, _NRBF_PAD - _NRBF), (0, 0))), (0, 2, 1))
    Wf2T = jnp.transpose(Wf2, (0, 2, 1))
    bf1c = bf1.reshape(_NB, _F, 1)
    bf2c = bf2.reshape(_NB, _F, 1)
    idx2 = jnp.stack([idx_j.astype(jnp.int32).reshape(-1, _CHUNK),
                      idx_i.astype(jnp.int32).reshape(-1, _CHUNK)], axis=1)

    x = _embed(Zc, emb, W_pe)
    W_edges = _filters(RijT, Wf1T, bf1c, Wf2T, bf2c)
    for b in range(_NB):
        xf = _project(x, W_in2f[b])
        agg_parts = _sc_agg(xf, W_edges[b], idx2)
        x = _node_update(agg_parts, x, Wo1[b], Wo2[b])
    return x


# speedup vs baseline: 1.8295x; 1.2013x over previous
import functools
import math

import jax
import jax.numpy as jnp
from jax import lax
from jax.experimental import pallas as pl
from jax.experimental.pallas import tpu as pltpu
from jax.experimental.pallas import tpu_sc as plsc

_N = 10000
_E = 320000
_F = 128
_NRBF = 20
_NRBF_PAD = 32
_NB = 3
_CUTOFF = 5.0
_MAXZ = 100
_LN2 = math.log(2.0)
_DELTA = _CUTOFF / (_NRBF - 1)
_COEFF = -0.5 / _DELTA ** 2

_NC = 2
_NS = 16
_NW = _NC * _NS
_EW = _E // _NW
_CHUNK = 40
_NCHUNK = _EW // _CHUNK
_NPAD = 10240
_ROWS_PER_TILE = _NPAD // _NS


def _ssp(x):
    return jnp.maximum(x, 0.0) + jnp.log(1.0 + jnp.exp(-jnp.abs(x))) - _LN2


def _pack_bf16(x):
    xi = lax.bitcast_convert_type(x[:, :_F // 2], jnp.int32)
    yi = lax.bitcast_convert_type(x[:, _F // 2:], jnp.int32)
    lo = lax.shift_right_logical(xi + 32768, 16)
    hi = lax.bitwise_and(yi + 32768, jnp.int32(-65536))
    return lax.bitwise_or(lo, hi)


_NBLK = 1000


def _embed_body(z_ref, emb_ref, wpe_ref, out_ref):
    z = z_ref[...]
    cols = lax.broadcasted_iota(jnp.int32, (_NBLK, _MAXZ), 1)
    oh = (z == cols).astype(jnp.float32)
    x0 = jnp.dot(oh, emb_ref[...], preferred_element_type=jnp.float32)
    out_ref[...] = jnp.dot(x0, wpe_ref[...],
                           preferred_element_type=jnp.float32)


def _embed(Zc, emb, W_pe):
    return pl.pallas_call(
        _embed_body,
        grid=(_N // _NBLK,),
        in_specs=[
            pl.BlockSpec((_NBLK, 1), lambda i: (i, 0)),
            pl.BlockSpec((_MAXZ, _F), lambda i: (0, 0)),
            pl.BlockSpec((_F, _F), lambda i: (0, 0)),
        ],
        out_specs=pl.BlockSpec((_NBLK, _F), lambda i: (i, 0)),
        out_shape=jax.ShapeDtypeStruct((_N, _F), jnp.float32),
    )(Zc, emb, W_pe)


_EBLK = 2560


def _filter_body(rij_ref, wf1_ref, bf1_ref, wf2_ref, bf2_ref, out_ref):
    r = rij_ref[...]
    rr = r * r
    d2 = jnp.sum(rr, axis=0, keepdims=True)
    d = jnp.sqrt(d2)
    ks = lax.broadcasted_iota(jnp.int32, (_NRBF_PAD, _EBLK), 0).astype(
        jnp.float32)
    offs = jnp.where(ks < float(_NRBF), ks * _DELTA, 1.0e6)
    f = jnp.exp(_COEFF * (d - offs) ** 2)
    inside = (d < _CUTOFF).astype(jnp.float32)
    rcut = 0.5 * (jnp.cos(d * (math.pi / _CUTOFF)) + 1.0) * inside
    fb = f.astype(jnp.bfloat16)
    half = _F // 2
    s1 = jnp.dot(wf1_ref[...].astype(jnp.bfloat16), fb,
                 preferred_element_type=jnp.float32)
    h = _ssp(s1 + bf1_ref[...]).astype(jnp.bfloat16)
    s2 = jnp.dot(wf2_ref[...].astype(jnp.bfloat16), h,
                 preferred_element_type=jnp.float32)
    wij = (s2 + bf2_ref[...]) * rcut
    xi = lax.bitcast_convert_type(wij[:half, :], jnp.int32)
    yi = lax.bitcast_convert_type(wij[half:, :], jnp.int32)
    lo = lax.shift_right_logical(xi + 32768, 16)
    hi = lax.bitwise_and(yi + 32768, jnp.int32(-65536))
    out_ref[...] = jnp.transpose(lax.bitwise_or(lo, hi))


def _filters_b(RijT, Wf1Tb, bf1cb, Wf2Tb, bf2cb):
    out_sd = jax.ShapeDtypeStruct((_E, _F // 2), jnp.int32)
    return pl.pallas_call(
        _filter_body,
        grid=(_E // _EBLK,),
        in_specs=[
            pl.BlockSpec((3, _EBLK), lambda i: (0, i)),
            pl.BlockSpec((_F, _NRBF_PAD), lambda i: (0, 0)),
            pl.BlockSpec((_F, 1), lambda i: (0, 0)),
            pl.BlockSpec((_F, _F), lambda i: (0, 0)),
            pl.BlockSpec((_F, 1), lambda i: (0, 0)),
        ],
        out_specs=pl.BlockSpec((_EBLK, _F // 2), lambda i: (i, 0)),
        out_shape=out_sd,
    )(RijT, Wf1Tb, bf1cb, Wf2Tb, bf2cb)


def _matmul_body(x_ref, w_ref, out_ref):
    out_ref[...] = jnp.dot(x_ref[...], w_ref[...],
                           preferred_element_type=jnp.float32)


_PBLK = 2000


def _project(x, W):
    return pl.pallas_call(
        _matmul_body,
        grid=(_N // _PBLK,),
        in_specs=[
            pl.BlockSpec((_PBLK, _F), lambda i: (i, 0)),
            pl.BlockSpec((_F, _F), lambda i: (0, 0)),
        ],
        out_specs=pl.BlockSpec((_PBLK, _F), lambda i: (i, 0)),
        out_shape=jax.ShapeDtypeStruct((_N, _F), jnp.float32),
    )(x, W)


def _update_body(agg_ref, x_ref, wo1_ref, wo2_ref, out_ref):
    agg = agg_ref[0] + agg_ref[1]
    h = _ssp(jnp.dot(agg, wo1_ref[...], preferred_element_type=jnp.float32))
    v = jnp.dot(h, wo2_ref[...], preferred_element_type=jnp.float32)
    out_ref[...] = x_ref[...] + v


def _node_update(agg_parts, x, Wo1, Wo2):
    return pl.pallas_call(
        _update_body,
        grid=(_N // _NBLK,),
        in_specs=[
            pl.BlockSpec((_NC, _NBLK, _F), lambda i: (0, i, 0)),
            pl.BlockSpec((_NBLK, _F), lambda i: (i, 0)),
            pl.BlockSpec((_F, _F), lambda i: (0, 0)),
            pl.BlockSpec((_F, _F), lambda i: (0, 0)),
        ],
        out_specs=pl.BlockSpec((_NBLK, _F), lambda i: (i, 0)),
        out_shape=jax.ShapeDtypeStruct((_N, _F), jnp.float32),
    )(agg_parts, x, Wo1, Wo2)


def _sc_agg_body(xf_hbm, w_hbm, idx2_hbm, out_hbm,
                 idx_c, rows_v, w_v, agg_sh,
                 sem_g, sem_w, sem_i, sem_s):
    cid = lax.axis_index("c")
    sid = lax.axis_index("s")
    wid = sid * _NC + cid
    ebase = wid * _EW

    zero16 = jnp.zeros((16,), jnp.float32)

    def _zb(i, carry):
        for k in range(_F // 16):
            rows_v[0, i, pl.ds(k * 16, 16)] = zero16
        return carry

    lax.fori_loop(0, _CHUNK, _zb, 0)
    for j in range(_ROWS_PER_TILE // _CHUNK):
        r0 = sid * _ROWS_PER_TILE + j * _CHUNK
        pltpu.sync_copy(rows_v.at[0], agg_sh.at[pl.ds(r0, _CHUNK)])
    plsc.subcore_barrier()

    def _wait_idx():
        pltpu.make_async_copy(idx2_hbm.at[0], idx_c.at[0], sem_i).wait()

    def _wait_rows(buf):
        pltpu.make_async_copy(xf_hbm.at[pl.ds(0, _CHUNK)],
                              rows_v.at[buf], sem_g).wait()

    def _wait_w(buf):
        pltpu.make_async_copy(w_hbm.at[pl.ds(0, _CHUNK)],
                              w_v.at[buf], sem_w).wait()

    def _wait_scatter(rb):
        pltpu.make_async_copy(rows_v.at[rb],
                              agg_sh.at[idx_c.at[0, 1]], sem_s).wait()

    def _fire_idx(ci, ib):
        pltpu.async_copy(idx2_hbm.at[wid * _NCHUNK + ci], idx_c.at[ib], sem_i)

    def _fire_data(ci, rb, ib):
        base = ebase + ci * _CHUNK
        pltpu.async_copy(xf_hbm.at[idx_c.at[ib, 0]], rows_v.at[rb], sem_g)
        pltpu.async_copy(w_hbm.at[pl.ds(base, _CHUNK)], w_v.at[rb], sem_w)

    himask = jnp.full((16,), -65536, jnp.int32)

    def _unpk(v):
        lo = lax.bitcast_convert_type(lax.shift_left(v, 16), jnp.float32)
        hi = lax.bitcast_convert_type(lax.bitwise_and(v, himask), jnp.float32)
        return lo, hi

    def _mul(rb):
        @plsc.parallel_loop(0, _CHUNK, unroll=2)
        def _body(e):
            for q in range(_F // 32):
                wlo, whi = _unpk(w_v[rb, e, pl.ds(q * 16, 16)])
                slo = pl.ds(q * 16, 16)
                shi = pl.ds(_F // 2 + q * 16, 16)
                rows_v[rb, e, slo] = rows_v[rb, e, slo] * wlo
                rows_v[rb, e, shi] = rows_v[rb, e, shi] * whi

    def _fire_scatter(rb, ib):
        pltpu.async_copy(rows_v.at[rb], agg_sh.at[idx_c.at[ib, 1]],
                         sem_s, add=True)

    _fire_idx(0, 0)
    _wait_idx()
    _fire_data(0, 0, 0)
    _fire_idx(1, 1)
    _wait_idx()
    _fire_data(1, 1, 1)
    _fire_idx(2, 2)
    _wait_idx()
    _fire_data(2, 2, 2)
    _fire_idx(3, 3)

    def _steady(c, carry):
        rb = lax.rem(c, 4)

        @pl.when(c > 0)
        def _():
            _wait_scatter(lax.rem(c + 3, 4))

        @pl.when(c <= _NCHUNK - 4)
        def _():
            _wait_idx()
            _fire_data(c + 3, lax.rem(c + 3, 4), lax.rem(c + 3, 5))

        @pl.when(c <= _NCHUNK - 5)
        def _():
            _fire_idx(c + 4, lax.rem(c + 4, 5))

        _wait_rows(rb)
        _wait_w(rb)
        _mul(rb)
        _fire_scatter(rb, lax.rem(c, 5))
        return carry

    lax.fori_loop(0, _NCHUNK, _steady, 0)
    _wait_scatter((_NCHUNK - 1) % 4)
    plsc.subcore_barrier()

    for j in range(_ROWS_PER_TILE // _CHUNK):
        r0 = sid * _ROWS_PER_TILE + j * _CHUNK
        pltpu.sync_copy(agg_sh.at[pl.ds(r0, _CHUNK)],
                        out_hbm.at[cid, pl.ds(r0, _CHUNK)])


@functools.cache
def _build_sc_agg():
    return functools.partial(
        pl.kernel,
        out_type=jax.ShapeDtypeStruct((_NC, _NPAD, _F), jnp.float32),
        mesh=plsc.VectorSubcoreMesh(core_axis_name="c", subcore_axis_name="s",
                                    num_cores=_NC, num_subcores=_NS),
        scratch_types=[
            pltpu.VMEM((5, 2, _CHUNK), jnp.int32),
            pltpu.VMEM((4, _CHUNK, _F), jnp.float32),
            pltpu.VMEM((4, _CHUNK, _F // 2), jnp.int32),
            pltpu.VMEM_SHARED((_NPAD, _F), jnp.float32),
            pltpu.SemaphoreType.DMA,
            pltpu.SemaphoreType.DMA,
            pltpu.SemaphoreType.DMA,
            pltpu.SemaphoreType.DMA,
        ],
    )(_sc_agg_body)


def _sc_agg(xf, w_edges, idx2):
    return _build_sc_agg()(xf, w_edges, idx2)


def kernel(Z, Rij, idx_i, idx_j, emb, W_pe, W_in2f, Wf1, bf1, Wf2, bf2,
           Wo1, Wo2):
    Zc = Z.astype(jnp.int32).reshape(_N, 1)
    RijT = Rij.T
    Wf1T = jnp.transpose(
        jnp.pad(Wf1, ((0, 0), (0, _NRBF_PAD - _NRBF), (0, 0))), (0, 2, 1))
    Wf2T = jnp.transpose(Wf2, (0, 2, 1))
    bf1c = bf1.reshape(_NB, _F, 1)
    bf2c = bf2.reshape(_NB, _F, 1)
    idx2 = jnp.stack([idx_j.astype(jnp.int32).reshape(-1, _CHUNK),
                      idx_i.astype(jnp.int32).reshape(-1, _CHUNK)], axis=1)

    x = _embed(Zc, emb, W_pe)
    W_edges = [_filters_b(RijT, Wf1T[b], bf1c[b], Wf2T[b], bf2c[b])
               for b in range(_NB)]
    for b in range(_NB):
        xf = _project(x, W_in2f[b])
        agg_parts = _sc_agg(xf, W_edges[b], idx2)
        x = _node_update(agg_parts, x, Wo1[b], Wo2[b])
    return x
